# Initial kernel scaffold; baseline (speedup 1.0000x reference)
import jax
import jax.numpy as jnp
from jax.experimental import pallas as pl


def kernel(atom, nbr, idx, crys_idx, atom2, nbr2, idx2, crys_idx2, s_vector, l_vector, mono_target1, mono_target2, W_emb, b_emb, convW, convb, bn1g, bn1b, bn2g, bn2b, W_fc, b_fc, W_emb2, b_emb2, convW2, convb2, bn1g2, bn1b2, bn2g2, bn2b2, W_fc2, b_fc2, W_fus, b_fus, W_out, b_out):
    raise NotImplementedError("write your pallas kernel here")


# speedup vs baseline: 2.4384x; 2.4384x over previous
import functools

import jax
import jax.numpy as jnp
from jax import lax
from jax.experimental import pallas as pl
from jax.experimental.pallas import tpu as pltpu
from jax.experimental.pallas import tpu_sc as plsc

AFD = 128
GFD = 256
NBRF = 41
MNB = 16
NAT = 10000
NEDGE = NAT * MNB
NCRY = 16
KCRY = 625
KPAD = 640

_SC_CORES = 2
_SC_SUBCORES = 16
_NW = _SC_CORES * _SC_SUBCORES


def _sc_gather(table, idx_flat, chunk):
    B = idx_flat.shape[0]
    D = table.shape[1]
    nchunks = B // chunk
    chunks_per_w = nchunks // _NW
    mesh = plsc.VectorSubcoreMesh(core_axis_name="c", subcore_axis_name="s")

    @functools.partial(
        pl.kernel,
        mesh=mesh,
        out_type=jax.ShapeDtypeStruct((B, D), table.dtype),
        scratch_types=[
            pltpu.VMEM((chunk,), jnp.int32),
            pltpu.VMEM((chunk, D), table.dtype),
            pltpu.SemaphoreType.DMA,
        ],
    )
    def k(table_hbm, idx_hbm, out_hbm, idx_v, rows_v, sem):
        wid = lax.axis_index("s") * _SC_CORES + lax.axis_index("c")

        @pl.loop(0, chunks_per_w)
        def _(c):
            base = (wid * chunks_per_w + c) * chunk
            pltpu.sync_copy(idx_hbm.at[pl.ds(base, chunk)], idx_v)
            pltpu.async_copy(table_hbm.at[idx_v], rows_v, sem).wait()
            pltpu.sync_copy(rows_v, out_hbm.at[pl.ds(base, chunk)])

    return k(table, idx_flat)


def _embed(x, W, b):
    N, F = x.shape
    TO = W.shape[1]
    TN = 2000

    def body(x_ref, w_ref, b_ref, o_ref):
        o_ref[...] = (
            jnp.dot(x_ref[...], w_ref[...], preferred_element_type=jnp.float32)
            + b_ref[...]
        )

    return pl.pallas_call(
        body,
        out_shape=jax.ShapeDtypeStruct((N, TO), jnp.float32),
        grid=(N // TN,),
        in_specs=[
            pl.BlockSpec((TN, F), lambda i: (i, 0)),
            pl.BlockSpec((F, TO), lambda i: (0, 0)),
            pl.BlockSpec((1, TO), lambda i: (0, 0)),
        ],
        out_specs=pl.BlockSpec((TN, TO), lambda i: (i, 0)),
    )(x, W, b.reshape(1, TO))


_T1 = 400
_E1 = _T1 * MNB


def _conv_pass1(a, G, nbr_flat, Wse, Wan, Wnbr):
    grid = NAT // _T1

    def body(a_ref, g_ref, n_ref, wse_ref, wan_ref, wnbr_ref, st_ref):
        i = pl.program_id(0)
        p = jnp.dot(g_ref[...], wan_ref[...], preferred_element_type=jnp.float32)
        p = p + jnp.dot(n_ref[...], wnbr_ref[...], preferred_element_type=jnp.float32)
        u = jnp.dot(a_ref[...], wse_ref[...], preferred_element_type=jnp.float32)
        pm = jnp.sum(p.reshape(_T1, MNB, GFD), axis=1)
        s0 = jnp.sum(p, axis=0, keepdims=True) + float(MNB) * jnp.sum(
            u, axis=0, keepdims=True
        )
        s1 = (
            jnp.sum(p * p, axis=0, keepdims=True)
            + 2.0 * jnp.sum(pm * u, axis=0, keepdims=True)
            + float(MNB) * jnp.sum(u * u, axis=0, keepdims=True)
        )
        vals = jnp.concatenate([s0, s1], axis=0)

        @pl.when(i == 0)
        def _():
            st_ref[...] = vals

        @pl.when(i > 0)
        def _():
            st_ref[...] += vals

    return pl.pallas_call(
        body,
        out_shape=jax.ShapeDtypeStruct((2, GFD), jnp.float32),
        grid=(grid,),
        in_specs=[
            pl.BlockSpec((_T1, AFD), lambda i: (i, 0)),
            pl.BlockSpec((_E1, AFD), lambda i: (i, 0)),
            pl.BlockSpec((_E1, NBRF), lambda i: (i, 0)),
            pl.BlockSpec((AFD, GFD), lambda i: (0, 0)),
            pl.BlockSpec((AFD, GFD), lambda i: (0, 0)),
            pl.BlockSpec((NBRF, GFD), lambda i: (0, 0)),
        ],
        out_specs=pl.BlockSpec((2, GFD), lambda i: (0, 0)),
    )(a, G, nbr_flat, Wse, Wan, Wnbr)


def _conv_pass2(a, G, nbr_flat, Wse, Wan, Wnbr, stats, g1, b1):
    grid = NAT // _T1

    def body(
        a_ref, g_ref, n_ref, wse_ref, wan_ref, wnbr_ref, st_ref, g1_ref, b1_ref,
        s_ref, ss_ref,
    ):
        i = pl.program_id(0)
        cnt = float(NEDGE)
        mean = st_ref[0:1, :] / cnt
        var = st_ref[1:2, :] / cnt - mean * mean
        scale = g1_ref[...] * lax.rsqrt(var + 1e-5)
        shift = b1_ref[...] - mean * scale
        p = jnp.dot(g_ref[...], wan_ref[...], preferred_element_type=jnp.float32)
        p = p + jnp.dot(n_ref[...], wnbr_ref[...], preferred_element_type=jnp.float32)
        u = jnp.dot(a_ref[...], wse_ref[...], preferred_element_type=jnp.float32)
        z = p.reshape(_T1, MNB, GFD) + u[:, None, :]
        z = z * scale.reshape(1, 1, GFD) + shift.reshape(1, 1, GFD)
        filt = jax.nn.sigmoid(z[..., :AFD])
        core = jax.nn.softplus(z[..., AFD:])
        s_t = jnp.sum(filt * core, axis=1)
        s_ref[...] = s_t
        v0 = jnp.sum(s_t, axis=0, keepdims=True)
        v1 = jnp.sum(s_t * s_t, axis=0, keepdims=True)
        vals = jnp.concatenate([v0, v1], axis=0)

        @pl.when(i == 0)
        def _():
            ss_ref[...] = vals

        @pl.when(i > 0)
        def _():
            ss_ref[...] += vals

    return pl.pallas_call(
        body,
        out_shape=[
            jax.ShapeDtypeStruct((NAT, AFD), jnp.float32),
            jax.ShapeDtypeStruct((2, AFD), jnp.float32),
        ],
        grid=(grid,),
        in_specs=[
            pl.BlockSpec((_T1, AFD), lambda i: (i, 0)),
            pl.BlockSpec((_E1, AFD), lambda i: (i, 0)),
            pl.BlockSpec((_E1, NBRF), lambda i: (i, 0)),
            pl.BlockSpec((AFD, GFD), lambda i: (0, 0)),
            pl.BlockSpec((AFD, GFD), lambda i: (0, 0)),
            pl.BlockSpec((NBRF, GFD), lambda i: (0, 0)),
            pl.BlockSpec((2, GFD), lambda i: (0, 0)),
            pl.BlockSpec((1, GFD), lambda i: (0, 0)),
            pl.BlockSpec((1, GFD), lambda i: (0, 0)),
        ],
        out_specs=[
            pl.BlockSpec((_T1, AFD), lambda i: (i, 0)),
            pl.BlockSpec((2, AFD), lambda i: (0, 0)),
        ],
    )(a, G, nbr_flat, Wse, Wan, Wnbr, stats, g1.reshape(1, GFD), b1.reshape(1, GFD))


def _conv_update(a, s, sstats, g2, b2):
    TN = 2000

    def body(a_ref, s_ref, ss_ref, g2_ref, b2_ref, o_ref):
        cnt = float(NAT)
        mean = ss_ref[0:1, :] / cnt
        var = ss_ref[1:2, :] / cnt - mean * mean
        scale = g2_ref[...] * lax.rsqrt(var + 1e-5)
        shift = b2_ref[...] - mean * scale
        o_ref[...] = jax.nn.softplus(a_ref[...] + s_ref[...] * scale + shift)

    return pl.pallas_call(
        body,
        out_shape=jax.ShapeDtypeStruct((NAT, AFD), jnp.float32),
        grid=(NAT // TN,),
        in_specs=[
            pl.BlockSpec((TN, AFD), lambda i: (i, 0)),
            pl.BlockSpec((TN, AFD), lambda i: (i, 0)),
            pl.BlockSpec((2, AFD), lambda i: (0, 0)),
            pl.BlockSpec((1, AFD), lambda i: (0, 0)),
            pl.BlockSpec((1, AFD), lambda i: (0, 0)),
        ],
        out_specs=pl.BlockSpec((TN, AFD), lambda i: (i, 0)),
    )(a, s, sstats, g2.reshape(1, AFD), b2.reshape(1, AFD))


def _final(gp1, gp2, m1, m2, sv, Wfc1, bfc1, Wfc2, bfc2, Wfus, bfus, Wout, bout):
    FI = Wfus.shape[0]

    def body(
        gp1_ref, gp2_ref, m1_ref, m2_ref, sv_ref, wfc1_ref, bfc1_ref, wfc2_ref,
        bfc2_ref, wfus_ref, bfus_ref, wout_ref, bout_ref, o_ref,
    ):
        def pool(gp_ref, wfc_ref, bfc_ref):
            x = gp_ref[...].reshape(NCRY, KPAD, AFD)
            iot = lax.broadcasted_iota(jnp.int32, (NCRY, KPAD, AFD), 1)
            x = jnp.where(iot < KCRY, x, 0.0)
            pooled = jnp.sum(x, axis=1) / float(KCRY)
            return (
                jnp.dot(pooled, wfc_ref[...], preferred_element_type=jnp.float32)
                + bfc_ref[...]
            )

        e1 = pool(gp1_ref, wfc1_ref, bfc1_ref)
        e2 = pool(gp2_ref, wfc2_ref, bfc2_ref)
        fused = jnp.concatenate(
            [e1, e2, m1_ref[...], m2_ref[...], sv_ref[...]], axis=1
        )
        h = jnp.dot(fused, wfus_ref[...], preferred_element_type=jnp.float32)
        h = jnp.maximum(h + bfus_ref[...], 0.0)
        o_ref[...] = (
            jnp.dot(h, wout_ref[...], preferred_element_type=jnp.float32)
            + bout_ref[...]
        )

    H = Wfc1.shape[1]
    return pl.pallas_call(
        body,
        out_shape=jax.ShapeDtypeStruct((NCRY, 1), jnp.float32),
        in_specs=[pl.BlockSpec(x.shape, lambda: tuple(0 for _ in x.shape))
                  for x in (gp1, gp2, m1, m2, sv)]
        + [
            pl.BlockSpec((AFD, H), lambda: (0, 0)),
            pl.BlockSpec((1, H), lambda: (0, 0)),
            pl.BlockSpec((AFD, H), lambda: (0, 0)),
            pl.BlockSpec((1, H), lambda: (0, 0)),
            pl.BlockSpec((FI, FI), lambda: (0, 0)),
            pl.BlockSpec((1, FI), lambda: (0, 0)),
            pl.BlockSpec((FI, 1), lambda: (0, 0)),
            pl.BlockSpec((1, 1), lambda: (0, 0)),
        ],
        out_specs=pl.BlockSpec((NCRY, 1), lambda: (0, 0)),
    )(
        gp1, gp2, m1, m2, sv,
        Wfc1, bfc1.reshape(1, H), Wfc2, bfc2.reshape(1, H),
        Wfus, bfus.reshape(1, FI), Wout, bout.reshape(1, 1),
    )


def _encode(atom_raw, nbr, idx, crys_idx, We, be, convW, g1s, b1s, g2s, b2s):
    a = _embed(atom_raw, We, be)
    nbr_flat = nbr.reshape(NEDGE, NBRF)
    idx_flat = idx.reshape(NEDGE).astype(jnp.int32)
    for i in range(convW.shape[0]):
        Wse = convW[i, :AFD]
        Wan = convW[i, AFD : 2 * AFD]
        Wnbr = convW[i, 2 * AFD :]
        G = _sc_gather(a, idx_flat, chunk=200)
        stats = _conv_pass1(a, G, nbr_flat, Wse, Wan, Wnbr)
        s, sstats = _conv_pass2(a, G, nbr_flat, Wse, Wan, Wnbr, stats, g1s[i], b1s[i])
        a = _conv_update(a, s, sstats, g2s[i], b2s[i])
    crys_flat = jnp.concatenate(
        [
            crys_idx.reshape(NCRY, KCRY),
            jnp.zeros((NCRY, KPAD - KCRY), crys_idx.dtype),
        ],
        axis=1,
    ).reshape(NCRY * KPAD).astype(jnp.int32)
    return _sc_gather(a, crys_flat, chunk=320)


def kernel(atom, nbr, idx, crys_idx, atom2, nbr2, idx2, crys_idx2, s_vector,
           l_vector, mono_target1, mono_target2, W_emb, b_emb, convW, convb,
           bn1g, bn1b, bn2g, bn2b, W_fc, b_fc, W_emb2, b_emb2, convW2, convb2,
           bn1g2, bn1b2, bn2g2, bn2b2, W_fc2, b_fc2, W_fus, b_fus, W_out, b_out):
    gp1 = _encode(atom, nbr, idx, crys_idx, W_emb, b_emb, convW, bn1g, bn1b,
                  bn2g, bn2b)
    gp2 = _encode(atom2, nbr2, idx2, crys_idx2, W_emb2, b_emb2, convW2, bn1g2,
                  bn1b2, bn2g2, bn2b2)
    return _final(gp1, gp2, mono_target1, mono_target2, s_vector, W_fc, b_fc,
                  W_fc2, b_fc2, W_fus, b_fus, W_out, b_out)
